# Initial kernel scaffold; baseline (speedup 1.0000x reference)
import jax
import jax.numpy as jnp
from jax.experimental import pallas as pl


def kernel(x, edge_index, edge_weight, W, b):
    raise NotImplementedError("write your pallas kernel here")


# speedup vs baseline: 4.8319x; 4.8319x over previous
import functools

import jax
import jax.numpy as jnp
from jax import lax
from jax.experimental import pallas as pl
from jax.experimental.pallas import tpu as pltpu
from jax.experimental.pallas import tpu_sc as plsc

NC = 2
NS = 16
LANES = 16

K = 128
ZR = 80


def _sc_body(n_nodes, n_edges, d,
             x_hbm, row_hbm, col_hbm, w_hbm, out_hbm,
             acc, zbuf, colv, rowv, wv, msgs, sem):
  cid = lax.axis_index("c")
  sid = lax.axis_index("s")
  wid = sid * NC + cid

  nvec = d // LANES
  zvec = jnp.zeros((LANES,), jnp.float32)

  def zfill(i, _):
    for j in range(nvec):
      zbuf[i, pl.ds(j * LANES, LANES)] = zvec
    return 0
  lax.fori_loop(0, ZR, zfill, 0)

  rows_t = jnp.where(sid < NS - 1, 640, n_nodes - 640 * (NS - 1))
  rbase = sid * 640

  def zcopy(i, _):
    pltpu.sync_copy(zbuf, acc.at[pl.ds(rbase + i * ZR, ZR)])
    return 0
  lax.fori_loop(0, rows_t // ZR, zcopy, 0)

  plsc.subcore_barrier()

  nb = n_edges // K
  nt = (nb - wid + (NC * NS) - 1) // (NC * NS)

  def batch(t, _):
    e0 = (wid + t * NC * NS) * K
    pltpu.sync_copy(col_hbm.at[pl.ds(e0, K)], colv)
    pltpu.sync_copy(row_hbm.at[pl.ds(e0, K)], rowv)
    pltpu.sync_copy(w_hbm.at[pl.ds(e0, K)], wv)
    pltpu.async_copy(x_hbm.at[colv], msgs, sem).wait()

    def scale(k, _):
      wspl = plsc.load_gather(wv, [jnp.full((LANES,), k, jnp.int32)])
      for j in range(nvec):
        sl = (k, pl.ds(j * LANES, LANES))
        msgs[sl] = msgs[sl] * wspl
      return 0
    lax.fori_loop(0, K, scale, 0)

    pltpu.sync_copy(msgs, acc.at[rowv], add=True)
    return 0
  lax.fori_loop(0, nt, batch, 0)

  plsc.subcore_barrier()

  pltpu.sync_copy(acc.at[pl.ds(rbase, rows_t)],
                  out_hbm.at[cid, pl.ds(rbase, rows_t)])


def _sc_scatter(x, row, col, w):
  n, d = x.shape
  e = row.shape[0]
  mesh = plsc.VectorSubcoreMesh(core_axis_name="c", subcore_axis_name="s")
  body = functools.partial(_sc_body, n, e, d)
  return pl.kernel(
      body,
      out_type=jax.ShapeDtypeStruct((NC, n, d), jnp.float32),
      mesh=mesh,
      compiler_params=pltpu.CompilerParams(needs_layout_passes=False),
      scratch_types=[
          pltpu.VMEM_SHARED((n, d), jnp.float32),
          pltpu.VMEM((ZR, d), jnp.float32),
          pltpu.VMEM((K,), jnp.int32),
          pltpu.VMEM((K,), jnp.int32),
          pltpu.VMEM((K,), jnp.float32),
          pltpu.VMEM((K, d), jnp.float32),
          pltpu.SemaphoreType.DMA,
      ],
  )(x, row, col, w)


def _tc_linear_body(p0_ref, p1_ref, w_ref, b_ref, o_ref):
  acc = p0_ref[0] + p1_ref[0]
  y = lax.dot_general(acc, w_ref[...], (((1,), (1,)), ((), ())),
                      preferred_element_type=jnp.float32)
  o_ref[...] = y + b_ref[...]


def _tc_linear(partials, W, b):
  _, n, d = partials.shape
  blk = 1000
  grid = (n // blk,)
  return pl.pallas_call(
      _tc_linear_body,
      grid=grid,
      in_specs=[
          pl.BlockSpec((1, blk, d), lambda i: (0, i, 0)),
          pl.BlockSpec((1, blk, d), lambda i: (1, i, 0)),
          pl.BlockSpec((d, d), lambda i: (0, 0)),
          pl.BlockSpec((1, d), lambda i: (0, 0)),
      ],
      out_specs=pl.BlockSpec((blk, d), lambda i: (i, 0)),
      out_shape=jax.ShapeDtypeStruct((n, d), jnp.float32),
  )(partials, partials, W, b[None, :])


@jax.jit
def kernel(x, edge_index, edge_weight, W, b):
  row = edge_index[0].astype(jnp.int32)
  col = edge_index[1].astype(jnp.int32)
  partials = _sc_scatter(x, row, col, edge_weight.astype(jnp.float32))
  return _tc_linear(partials, W, b)


# speedup vs baseline: 7.9451x; 1.6443x over previous
import functools

import jax
import jax.numpy as jnp
from jax import lax
from jax.experimental import pallas as pl
from jax.experimental.pallas import tpu as pltpu
from jax.experimental.pallas import tpu_sc as plsc

NC = 2
NS = 16
NW = NC * NS
LANES = 16

K = 128
ZR = 16
ROWS_MAIN = 640
ROW_I = 0
COL_I = 1
W_I = 2


def _sc_body(n_nodes, nb, d,
             x_hbm, ed_hbm, out_hbm,
             acc, zbuf, ed0, ed1, msgs0, msgs1,
             gsem0, gsem1, wsem0, wsem1):
  cid = lax.axis_index("c")
  sid = lax.axis_index("s")
  wid = sid * NC + cid
  nvec = d // LANES
  nfull = nb // NW

  zvec = jnp.zeros((LANES,), jnp.float32)

  def zfill(i, _):
    for j in range(nvec):
      zbuf[i, pl.ds(j * LANES, LANES)] = zvec
    return 0
  lax.fori_loop(0, ZR, zfill, 0)

  rows_t = jnp.where(sid < NS - 1, ROWS_MAIN, n_nodes - ROWS_MAIN * (NS - 1))
  rbase = sid * ROWS_MAIN

  def zcopy(i, _):
    pltpu.sync_copy(zbuf, acc.at[pl.ds(rbase + i * ZR, ZR)])
    return 0
  lax.fori_loop(0, rows_t // ZR, zcopy, 0)

  plsc.subcore_barrier()

  def load_idx(b, ed):
    pltpu.sync_copy(ed_hbm.at[b], ed)

  def start_gather(ed, msgs, gsem):
    pltpu.async_copy(x_hbm.at[ed.at[COL_I]], msgs, gsem)

  def wait_gather(ed, msgs, gsem):
    pltpu.make_async_copy(x_hbm.at[ed.at[COL_I]], msgs, gsem).wait()

  def start_scatter(ed, msgs, wsem):
    pltpu.async_copy(msgs, acc.at[ed.at[ROW_I]], wsem, add=True)

  def wait_scatter(ed, msgs, wsem):
    pltpu.make_async_copy(msgs, acc.at[ed.at[ROW_I]], wsem).wait()

  def scale(ed, msgs):
    def sbody(u, _):
      for q in range(4):
        k = u * 4 + q
        wbits = plsc.load_gather(
            ed, [jnp.full((LANES,), W_I, jnp.int32),
                 jnp.full((LANES,), k, jnp.int32)])
        wspl = plsc.bitcast(wbits, jnp.float32)
        for j in range(nvec):
          sl = (k, pl.ds(j * LANES, LANES))
          msgs[sl] = msgs[sl] * wspl
      return 0
    lax.fori_loop(0, K // 4, sbody, 0)

  load_idx(wid, ed0)
  start_gather(ed0, msgs0, gsem0)

  def pair(i, _):
    t0 = 2 * i
    b0 = wid + NW * t0
    @pl.when(i > 0)
    def _():
      wait_scatter(ed1, msgs1, wsem1)
    load_idx(b0 + NW, ed1)
    start_gather(ed1, msgs1, gsem1)
    wait_gather(ed0, msgs0, gsem0)
    scale(ed0, msgs0)
    start_scatter(ed0, msgs0, wsem0)
    wait_scatter(ed0, msgs0, wsem0)

    @pl.when(t0 + 2 < nfull)
    def _():
      load_idx(b0 + 2 * NW, ed0)
      start_gather(ed0, msgs0, gsem0)
    wait_gather(ed1, msgs1, gsem1)
    scale(ed1, msgs1)
    start_scatter(ed1, msgs1, wsem1)
    return 0
  lax.fori_loop(0, nfull // 2, pair, 0)

  wait_scatter(ed1, msgs1, wsem1)

  @pl.when(nfull * NW + wid < nb)
  def _():
    load_idx(nfull * NW + wid, ed0)
    pltpu.async_copy(x_hbm.at[ed0.at[COL_I]], msgs0, gsem0).wait()
    scale(ed0, msgs0)
    pltpu.async_copy(msgs0, acc.at[ed0.at[ROW_I]], wsem0, add=True).wait()

  plsc.subcore_barrier()

  def drain(i, _):
    pltpu.sync_copy(acc.at[pl.ds(rbase + i * ZR, ZR)],
                    out_hbm.at[cid, pl.ds(rbase + i * ZR, ZR)])
    return 0
  lax.fori_loop(0, rows_t // ZR, drain, 0)


def _sc_scatter(x, edata):
  n, d = x.shape
  nb = edata.shape[0]
  mesh = plsc.VectorSubcoreMesh(core_axis_name="c", subcore_axis_name="s")
  body = functools.partial(_sc_body, n, nb, d)
  return pl.kernel(
      body,
      out_type=jax.ShapeDtypeStruct((NC, n, d), jnp.float32),
      mesh=mesh,
      compiler_params=pltpu.CompilerParams(needs_layout_passes=False),
      scratch_types=[
          pltpu.VMEM_SHARED((n, d), jnp.float32),
          pltpu.VMEM((ZR, d), jnp.float32),
          pltpu.VMEM((3, K), jnp.int32),
          pltpu.VMEM((3, K), jnp.int32),
          pltpu.VMEM((K, d), jnp.float32),
          pltpu.VMEM((K, d), jnp.float32),
          pltpu.SemaphoreType.DMA,
          pltpu.SemaphoreType.DMA,
          pltpu.SemaphoreType.DMA,
          pltpu.SemaphoreType.DMA,
      ],
  )(x, edata)


def _tc_linear_body(p0_ref, p1_ref, w_ref, b_ref, o_ref):
  acc = p0_ref[0] + p1_ref[0]
  y = lax.dot_general(acc, w_ref[...], (((1,), (1,)), ((), ())),
                      preferred_element_type=jnp.float32)
  o_ref[...] = y + b_ref[...]


def _tc_linear(partials, W, b):
  _, n, d = partials.shape
  blk = 1000
  grid = (n // blk,)
  return pl.pallas_call(
      _tc_linear_body,
      grid=grid,
      in_specs=[
          pl.BlockSpec((1, blk, d), lambda i: (0, i, 0)),
          pl.BlockSpec((1, blk, d), lambda i: (1, i, 0)),
          pl.BlockSpec((d, d), lambda i: (0, 0)),
          pl.BlockSpec((1, d), lambda i: (0, 0)),
      ],
      out_specs=pl.BlockSpec((blk, d), lambda i: (i, 0)),
      out_shape=jax.ShapeDtypeStruct((n, d), jnp.float32),
  )(partials, partials, W, b[None, :])


@jax.jit
def kernel(x, edge_index, edge_weight, W, b):
  e = edge_weight.shape[0]
  row = edge_index[0].astype(jnp.int32).reshape(e // K, K)
  col = edge_index[1].astype(jnp.int32).reshape(e // K, K)
  wbits = lax.bitcast_convert_type(
      edge_weight.astype(jnp.float32), jnp.int32).reshape(e // K, K)
  edata = jnp.stack([row, col, wbits], axis=1)
  partials = _sc_scatter(x, edata)
  return _tc_linear(partials, W, b)


# speedup vs baseline: 10.5736x; 1.3308x over previous
import functools

import jax
import jax.numpy as jnp
from jax import lax
from jax.experimental import pallas as pl
from jax.experimental.pallas import tpu as pltpu
from jax.experimental.pallas import tpu_sc as plsc

NC = 2
NS = 16
NW = NC * NS
LANES = 16

K = 80
NBUF = 4
ROWS_MAIN = 640
ROW_I = 0
COL_I = 1
W_I = 2


def _sc_body(n_nodes, nb, d,
             x_hbm, ed_hbm, out_hbm,
             acc, eds, msgss, isems, gsems, wsems):
  cid = lax.axis_index("c")
  sid = lax.axis_index("s")
  wid = sid * NC + cid
  nvec = d // LANES
  nfull = nb // NW

  zvec = jnp.zeros((LANES,), jnp.float32)
  zbuf = msgss[0]

  def zfill(i, _):
    for j in range(nvec):
      zbuf[i, pl.ds(j * LANES, LANES)] = zvec
    return 0
  lax.fori_loop(0, K, zfill, 0)

  rows_t = jnp.where(sid < NS - 1, ROWS_MAIN, n_nodes - ROWS_MAIN * (NS - 1))
  rbase = sid * ROWS_MAIN

  def zcopy(i, _):
    pltpu.sync_copy(zbuf, acc.at[pl.ds(rbase + i * K, K)])
    return 0
  lax.fori_loop(0, rows_t // K, zcopy, 0)

  plsc.subcore_barrier()

  def bid(t):
    return wid + NW * t

  def start_idx(t, s):
    pltpu.async_copy(ed_hbm.at[bid(t)], eds[s], isems[s])

  def wait_idx(t, s):
    pltpu.make_async_copy(ed_hbm.at[bid(t)], eds[s], isems[s]).wait()

  def start_gather(s):
    pltpu.async_copy(x_hbm.at[eds[s].at[COL_I]], msgss[s], gsems[s])

  def wait_gather(s):
    pltpu.make_async_copy(x_hbm.at[eds[s].at[COL_I]], msgss[s],
                          gsems[s]).wait()

  def start_scatter(s):
    pltpu.async_copy(msgss[s], acc.at[eds[s].at[ROW_I]], wsems[s], add=True)

  def wait_scatter(s):
    pltpu.make_async_copy(msgss[s], acc.at[eds[s].at[ROW_I]],
                          wsems[s]).wait()

  def scale(s):
    ed = eds[s]
    msgs = msgss[s]

    def sbody(u, _):
      for q in range(4):
        k = u * 4 + q
        wbits = plsc.load_gather(
            ed, [jnp.full((LANES,), W_I, jnp.int32),
                 jnp.full((LANES,), k, jnp.int32)])
        wspl = plsc.bitcast(wbits, jnp.float32)
        for j in range(nvec):
          sl = (k, pl.ds(j * LANES, LANES))
          msgs[sl] = msgs[sl] * wspl
      return 0
    lax.fori_loop(0, K // 4, sbody, 0)

  def unit(t, u):
    s = u % NBUF
    s1 = (u + 1) % NBUF
    s2 = (u + 2) % NBUF

    @pl.when(t >= 2)
    def _():
      wait_scatter(s2)

    @pl.when(t + 2 < nfull)
    def _():
      start_idx(t + 2, s2)

    @pl.when(t + 1 < nfull)
    def _():
      @pl.when(t >= 1)
      def _():
        wait_idx(t + 1, s1)
      start_gather(s1)
    wait_gather(s)
    scale(s)
    start_scatter(s)

  pltpu.sync_copy(ed_hbm.at[bid(0)], eds[0])
  pltpu.sync_copy(ed_hbm.at[bid(1)], eds[1])
  start_gather(0)

  def quad(i, _):
    t0 = NBUF * i
    for u in range(NBUF):
      unit(t0 + u, u)
    return 0
  lax.fori_loop(0, nfull // NBUF, quad, 0)

  for u in range(nfull % NBUF):
    unit(nfull - (nfull % NBUF) + u, u)

  wait_scatter((nfull - 2) % NBUF)
  wait_scatter((nfull - 1) % NBUF)

  plsc.subcore_barrier()

  def drain(i, _):
    pltpu.sync_copy(acc.at[pl.ds(rbase + i * K, K)],
                    out_hbm.at[cid, pl.ds(rbase + i * K, K)])
    return 0
  lax.fori_loop(0, rows_t // K, drain, 0)


def _sc_scatter(x, edata):
  n, d = x.shape
  nb = edata.shape[0]
  mesh = plsc.VectorSubcoreMesh(core_axis_name="c", subcore_axis_name="s")
  body = functools.partial(_sc_body, n, nb, d)
  return pl.kernel(
      body,
      out_type=jax.ShapeDtypeStruct((NC, n, d), jnp.float32),
      mesh=mesh,
      compiler_params=pltpu.CompilerParams(needs_layout_passes=False),
      scratch_types=[
          pltpu.VMEM_SHARED((n, d), jnp.float32),
          [pltpu.VMEM((3, K), jnp.int32)] * NBUF,
          [pltpu.VMEM((K, d), jnp.float32)] * NBUF,
          [pltpu.SemaphoreType.DMA] * NBUF,
          [pltpu.SemaphoreType.DMA] * NBUF,
          [pltpu.SemaphoreType.DMA] * NBUF,
      ],
  )(x, edata)


def _tc_linear_body(p0_ref, p1_ref, w_ref, b_ref, o_ref):
  acc = p0_ref[0] + p1_ref[0]
  y = lax.dot_general(acc, w_ref[...], (((1,), (1,)), ((), ())),
                      preferred_element_type=jnp.float32)
  o_ref[...] = y + b_ref[...]


def _tc_linear(partials, W, b):
  _, n, d = partials.shape
  blk = 1000
  grid = (n // blk,)
  return pl.pallas_call(
      _tc_linear_body,
      grid=grid,
      in_specs=[
          pl.BlockSpec((1, blk, d), lambda i: (0, i, 0)),
          pl.BlockSpec((1, blk, d), lambda i: (1, i, 0)),
          pl.BlockSpec((d, d), lambda i: (0, 0)),
          pl.BlockSpec((1, d), lambda i: (0, 0)),
      ],
      out_specs=pl.BlockSpec((blk, d), lambda i: (i, 0)),
      out_shape=jax.ShapeDtypeStruct((n, d), jnp.float32),
  )(partials, partials, W, b[None, :])


@jax.jit
def kernel(x, edge_index, edge_weight, W, b):
  e = edge_weight.shape[0]
  row = edge_index[0].astype(jnp.int32).reshape(e // K, K)
  col = edge_index[1].astype(jnp.int32).reshape(e // K, K)
  wbits = lax.bitcast_convert_type(
      edge_weight.astype(jnp.float32), jnp.int32).reshape(e // K, K)
  edata = jnp.stack([row, col, wbits], axis=1)
  partials = _sc_scatter(x, edata)
  return _tc_linear(partials, W, b)


# speedup vs baseline: 10.9383x; 1.0345x over previous
import functools

import jax
import jax.numpy as jnp
from jax import lax
from jax.experimental import pallas as pl
from jax.experimental.pallas import tpu as pltpu
from jax.experimental.pallas import tpu_sc as plsc

NC = 2
NS = 16
NW = NC * NS
LANES = 16

K = 80
NBUF = 4
ROWS_MAIN = 640
ROW_I = 0
COL_I = 1
W_I = 2


def _sc_body(n_nodes, nb, d,
             x_hbm, ed_hbm, out_hbm,
             acc, eds, msgss, isems, gsems, wsems):
  cid = lax.axis_index("c")
  sid = lax.axis_index("s")
  wid = sid * NC + cid
  nvec = d // LANES
  nfull = nb // NW

  zvec = jnp.zeros((LANES,), jnp.float32)
  zbuf = msgss[0]

  def zfill(i, _):
    for j in range(nvec):
      zbuf[i, pl.ds(j * LANES, LANES)] = zvec
    return 0
  lax.fori_loop(0, K, zfill, 0)

  rows_t = jnp.where(sid < NS - 1, ROWS_MAIN, n_nodes - ROWS_MAIN * (NS - 1))
  rbase = sid * ROWS_MAIN

  def zcopy(i, _):
    pltpu.sync_copy(zbuf, acc.at[pl.ds(rbase + i * K, K)])
    return 0
  lax.fori_loop(0, rows_t // K, zcopy, 0)

  plsc.subcore_barrier()

  def bid(t):
    return wid + NW * t

  def start_idx(t, s):
    pltpu.async_copy(ed_hbm.at[bid(t)], eds[s], isems[s])

  def wait_idx(t, s):
    pltpu.make_async_copy(ed_hbm.at[bid(t)], eds[s], isems[s]).wait()

  def start_gather(s):
    pltpu.async_copy(x_hbm.at[eds[s].at[COL_I]], msgss[s], gsems[s])

  def wait_gather(s):
    pltpu.make_async_copy(x_hbm.at[eds[s].at[COL_I]], msgss[s],
                          gsems[s]).wait()

  def start_scatter(s):
    pltpu.async_copy(msgss[s], acc.at[eds[s].at[ROW_I]], wsems[s], add=True)

  def wait_scatter(s):
    pltpu.make_async_copy(msgss[s], acc.at[eds[s].at[ROW_I]],
                          wsems[s]).wait()

  def scale(s):
    ed = eds[s]
    msgs = msgss[s]

    def sbody(g, _):
      w16 = plsc.bitcast(ed[W_I, pl.ds(g * LANES, LANES)], jnp.float32)
      for q in range(LANES):
        k = g * LANES + q
        wspl = w16.at[jnp.full((LANES,), q, jnp.int32)].get(
            mode="promise_in_bounds")
        for j in range(nvec):
          sl = (k, pl.ds(j * LANES, LANES))
          msgs[sl] = msgs[sl] * wspl
      return 0
    lax.fori_loop(0, K // LANES, sbody, 0)

  def unit(t, u):
    s = u % NBUF
    s1 = (u + 1) % NBUF
    s2 = (u + 2) % NBUF

    @pl.when(t >= 2)
    def _():
      wait_scatter(s2)

    @pl.when(t + 2 < nfull)
    def _():
      start_idx(t + 2, s2)

    @pl.when(t + 1 < nfull)
    def _():
      @pl.when(t >= 1)
      def _():
        wait_idx(t + 1, s1)
      start_gather(s1)
    wait_gather(s)
    scale(s)
    start_scatter(s)

  pltpu.sync_copy(ed_hbm.at[bid(0)], eds[0])
  pltpu.sync_copy(ed_hbm.at[bid(1)], eds[1])
  start_gather(0)

  def quad(i, _):
    t0 = NBUF * i
    for u in range(NBUF):
      unit(t0 + u, u)
    return 0
  lax.fori_loop(0, nfull // NBUF, quad, 0)

  for u in range(nfull % NBUF):
    unit(nfull - (nfull % NBUF) + u, u)

  wait_scatter((nfull - 2) % NBUF)
  wait_scatter((nfull - 1) % NBUF)

  plsc.subcore_barrier()

  def drain(i, _):
    pltpu.sync_copy(acc.at[pl.ds(rbase + i * K, K)],
                    out_hbm.at[cid, pl.ds(rbase + i * K, K)])
    return 0
  lax.fori_loop(0, rows_t // K, drain, 0)


def _sc_scatter(x, edata):
  n, d = x.shape
  nb = edata.shape[0]
  mesh = plsc.VectorSubcoreMesh(core_axis_name="c", subcore_axis_name="s")
  body = functools.partial(_sc_body, n, nb, d)
  return pl.kernel(
      body,
      out_type=jax.ShapeDtypeStruct((NC, n, d), jnp.float32),
      mesh=mesh,
      compiler_params=pltpu.CompilerParams(needs_layout_passes=False,
                                           use_tc_tiling_on_sc=False),
      scratch_types=[
          pltpu.VMEM_SHARED((n, d), jnp.float32),
          [pltpu.VMEM((3, K), jnp.int32)] * NBUF,
          [pltpu.VMEM((K, d), jnp.float32)] * NBUF,
          [pltpu.SemaphoreType.DMA] * NBUF,
          [pltpu.SemaphoreType.DMA] * NBUF,
          [pltpu.SemaphoreType.DMA] * NBUF,
      ],
  )(x, edata)


def _tc_linear_body(p0_ref, p1_ref, w_ref, b_ref, o_ref):
  acc = p0_ref[0] + p1_ref[0]
  y = lax.dot_general(acc, w_ref[...], (((1,), (1,)), ((), ())),
                      preferred_element_type=jnp.float32)
  o_ref[...] = y + b_ref[...]


def _tc_linear(partials, W, b):
  _, n, d = partials.shape
  blk = 1000
  grid = (n // blk,)
  return pl.pallas_call(
      _tc_linear_body,
      grid=grid,
      in_specs=[
          pl.BlockSpec((1, blk, d), lambda i: (0, i, 0)),
          pl.BlockSpec((1, blk, d), lambda i: (1, i, 0)),
          pl.BlockSpec((d, d), lambda i: (0, 0)),
          pl.BlockSpec((1, d), lambda i: (0, 0)),
      ],
      out_specs=pl.BlockSpec((blk, d), lambda i: (i, 0)),
      out_shape=jax.ShapeDtypeStruct((n, d), jnp.float32),
  )(partials, partials, W, b[None, :])


@jax.jit
def kernel(x, edge_index, edge_weight, W, b):
  e = edge_weight.shape[0]
  row = edge_index[0].astype(jnp.int32).reshape(e // K, K)
  col = edge_index[1].astype(jnp.int32).reshape(e // K, K)
  wbits = lax.bitcast_convert_type(
      edge_weight.astype(jnp.float32), jnp.int32).reshape(e // K, K)
  edata = jnp.stack([row, col, wbits], axis=1)
  partials = _sc_scatter(x, edata)
  return _tc_linear(partials, W, b)


# speedup vs baseline: 12.7369x; 1.1644x over previous
import functools

import jax
import jax.numpy as jnp
from jax import lax
from jax.experimental import pallas as pl
from jax.experimental.pallas import tpu as pltpu
from jax.experimental.pallas import tpu_sc as plsc

NC = 2
NS = 16
NW = NC * NS
LANES = 16

K = 80
NBUF = 4
ROWS_MAIN = 640


def _sc_body(n_nodes, nb, d,
             x_hbm, row_hbm, col_hbm, w_hbm, out_hbm,
             acc, colvs, rowvs, wvs, msgss, isems, gsems, wsems, dsem):
  cid = lax.axis_index("c")
  sid = lax.axis_index("s")
  wid = sid * NC + cid
  nvec = d // LANES
  nfull = nb // NW

  zvec = jnp.zeros((LANES,), jnp.float32)
  zbuf = msgss[0]

  def zfill(i, _):
    for j in range(nvec):
      zbuf[i, pl.ds(j * LANES, LANES)] = zvec
    return 0
  lax.fori_loop(0, K, zfill, 0)

  rows_t = jnp.where(sid < NS - 1, ROWS_MAIN, n_nodes - ROWS_MAIN * (NS - 1))
  rbase = sid * ROWS_MAIN

  def zstart(i, _):
    pltpu.async_copy(zbuf, acc.at[pl.ds(rbase + i * K, K)], dsem)
    return 0
  lax.fori_loop(0, rows_t // K, zstart, 0)

  def zwait(i, _):
    pltpu.make_async_copy(zbuf, acc.at[pl.ds(rbase + i * K, K)], dsem).wait()
    return 0
  lax.fori_loop(0, rows_t // K, zwait, 0)

  plsc.subcore_barrier()

  def bid(t):
    return wid + NW * t

  def start_idx(t, s):
    b = bid(t)
    pltpu.async_copy(row_hbm.at[b], rowvs[s], isems[s])
    pltpu.async_copy(col_hbm.at[b], colvs[s], isems[s])
    pltpu.async_copy(w_hbm.at[b], wvs[s], isems[s])

  def wait_idx(t, s):
    b = bid(t)
    pltpu.make_async_copy(row_hbm.at[b], rowvs[s], isems[s]).wait()
    pltpu.make_async_copy(col_hbm.at[b], colvs[s], isems[s]).wait()
    pltpu.make_async_copy(w_hbm.at[b], wvs[s], isems[s]).wait()

  def start_gather(s):
    pltpu.async_copy(x_hbm.at[colvs[s]], msgss[s], gsems[s])

  def wait_gather(s):
    pltpu.make_async_copy(x_hbm.at[colvs[s]], msgss[s], gsems[s]).wait()

  def start_scatter(s):
    pltpu.async_copy(msgss[s], acc.at[rowvs[s]], wsems[s], add=True)

  def wait_scatter(s):
    pltpu.make_async_copy(msgss[s], acc.at[rowvs[s]], wsems[s]).wait()

  def scale(s):
    wv = wvs[s]
    msgs = msgss[s]

    def sbody(g, _):
      w16 = wv[pl.ds(g * LANES, LANES)]
      for q in range(LANES):
        k = g * LANES + q
        wspl = w16.at[jnp.full((LANES,), q, jnp.int32)].get(
            mode="promise_in_bounds")
        for j in range(nvec):
          sl = (k, pl.ds(j * LANES, LANES))
          msgs[sl] = msgs[sl] * wspl
      return 0
    lax.fori_loop(0, K // LANES, sbody, 0)

  def unit(t, u):
    s = u % NBUF
    s1 = (u + 1) % NBUF
    s2 = (u + 2) % NBUF

    @pl.when(t >= 2)
    def _():
      wait_scatter(s2)

    @pl.when(t + 2 < nfull)
    def _():
      start_idx(t + 2, s2)

    @pl.when(t + 1 < nfull)
    def _():
      @pl.when(t >= 1)
      def _():
        wait_idx(t + 1, s1)
      start_gather(s1)
    wait_gather(s)
    scale(s)
    start_scatter(s)

  start_idx(0, 0)
  start_idx(1, 1)
  wait_idx(0, 0)
  wait_idx(1, 1)
  start_gather(0)

  def quad(i, _):
    t0 = NBUF * i
    for u in range(NBUF):
      unit(t0 + u, u)
    return 0
  lax.fori_loop(0, nfull // NBUF, quad, 0)

  for u in range(nfull % NBUF):
    unit(nfull - (nfull % NBUF) + u, u)

  wait_scatter((nfull - 2) % NBUF)
  wait_scatter((nfull - 1) % NBUF)

  plsc.subcore_barrier()

  def dstart(i, _):
    pltpu.async_copy(acc.at[pl.ds(rbase + i * K, K)],
                     out_hbm.at[cid, pl.ds(rbase + i * K, K)], dsem)
    return 0
  lax.fori_loop(0, rows_t // K, dstart, 0)

  def dwait(i, _):
    pltpu.make_async_copy(acc.at[pl.ds(rbase + i * K, K)],
                          out_hbm.at[cid, pl.ds(rbase + i * K, K)],
                          dsem).wait()
    return 0
  lax.fori_loop(0, rows_t // K, dwait, 0)


def _sc_scatter(x, row2d, col2d, w2d):
  n, d = x.shape
  nb = row2d.shape[0]
  mesh = plsc.VectorSubcoreMesh(core_axis_name="c", subcore_axis_name="s")
  body = functools.partial(_sc_body, n, nb, d)
  return pl.kernel(
      body,
      out_type=jax.ShapeDtypeStruct((NC, n, d), jnp.float32),
      mesh=mesh,
      compiler_params=pltpu.CompilerParams(needs_layout_passes=False,
                                           use_tc_tiling_on_sc=False),
      scratch_types=[
          pltpu.VMEM_SHARED((n, d), jnp.float32),
          [pltpu.VMEM((K,), jnp.int32)] * NBUF,
          [pltpu.VMEM((K,), jnp.int32)] * NBUF,
          [pltpu.VMEM((K,), jnp.float32)] * NBUF,
          [pltpu.VMEM((K, d), jnp.float32)] * NBUF,
          [pltpu.SemaphoreType.DMA] * NBUF,
          [pltpu.SemaphoreType.DMA] * NBUF,
          [pltpu.SemaphoreType.DMA] * NBUF,
          pltpu.SemaphoreType.DMA,
      ],
  )(x, row2d, col2d, w2d)


def _tc_linear_body(p0_ref, p1_ref, w_ref, b_ref, o_ref):
  acc = p0_ref[0] + p1_ref[0]
  y = lax.dot_general(acc, w_ref[...], (((1,), (1,)), ((), ())),
                      preferred_element_type=jnp.float32)
  o_ref[...] = y + b_ref[...]


def _tc_linear(partials, W, b):
  _, n, d = partials.shape
  blk = 1000
  grid = (n // blk,)
  return pl.pallas_call(
      _tc_linear_body,
      grid=grid,
      in_specs=[
          pl.BlockSpec((1, blk, d), lambda i: (0, i, 0)),
          pl.BlockSpec((1, blk, d), lambda i: (1, i, 0)),
          pl.BlockSpec((d, d), lambda i: (0, 0)),
          pl.BlockSpec((1, d), lambda i: (0, 0)),
      ],
      out_specs=pl.BlockSpec((blk, d), lambda i: (i, 0)),
      out_shape=jax.ShapeDtypeStruct((n, d), jnp.float32),
  )(partials, partials, W, b[None, :])


@jax.jit
def kernel(x, edge_index, edge_weight, W, b):
  e = edge_weight.shape[0]
  row = edge_index[0].astype(jnp.int32).reshape(e // K, K)
  col = edge_index[1].astype(jnp.int32).reshape(e // K, K)
  w2d = edge_weight.astype(jnp.float32).reshape(e // K, K)
  partials = _sc_scatter(x, row, col, w2d)
  return _tc_linear(partials, W, b)
